# baseline (speedup 1.0000x reference)
import functools

import jax
import jax.numpy as jnp
from jax import lax
from jax.experimental import pallas as pl
from jax.experimental.pallas import tpu as pltpu
from jax.experimental.pallas import tpu_sc as plsc

B = 1024
L = 200
WORD_DIM = 128
POS_SIZE = 16
OUT_DIM = WORD_DIM + 2 * POS_SIZE

NC, NS = 2, 16
NW = NC * NS
NBB = B // 128
NLR = NW // NBB
NCH = L // NLR
LBLK = 56


def _make_kernel():
    mesh = plsc.VectorSubcoreMesh(core_axis_name="c", subcore_axis_name="s")

    @functools.partial(
        pl.kernel,
        mesh=mesh,
        out_type=jax.ShapeDtypeStruct((L * OUT_DIM, B), jnp.float32),
        compiler_params=pltpu.CompilerParams(needs_layout_passes=False),
        scratch_types=[
            pltpu.VMEM((LBLK, 128), jnp.int32),
            pltpu.VMEM((LBLK, 128), jnp.int32),
            pltpu.VMEM((LBLK, 128), jnp.int32),
            pltpu.VMEM((64, 128), jnp.float32),
            pltpu.VMEM((64, 128), jnp.float32),
            pltpu.VMEM((2 * 128, 128), jnp.float32),
            pltpu.VMEM((2 * OUT_DIM, 128), jnp.float32),
            pltpu.SemaphoreType.DMA,
            pltpu.SemaphoreType.DMA,
        ],
    )
    def lookup(word_i, pos1_i, pos2_i, wtab, p1tab, p2tab, out,
               widx_v, p1idx_v, p2idx_v, p1tab_v, p2tab_v, wrows, obuf,
               gsem, wsem):
        wid = lax.axis_index("s") * NC + lax.axis_index("c")
        bb = lax.div(wid, NLR)
        lr = lax.rem(wid, NLR)
        col0 = bb * 128
        lread = lr * 48
        loff = lr * 2
        lanes = lax.iota(jnp.int32, 16)

        pltpu.sync_copy(word_i.at[pl.ds(lread, LBLK), pl.ds(col0, 128)],
                        widx_v)
        pltpu.sync_copy(pos1_i.at[pl.ds(lread, LBLK), pl.ds(col0, 128)],
                        p1idx_v)
        pltpu.sync_copy(pos2_i.at[pl.ds(lread, LBLK), pl.ds(col0, 128)],
                        p2idx_v)
        pltpu.sync_copy(p1tab, p1tab_v)
        pltpu.sync_copy(p2tab, p2tab_v)

        def fire_gather(j, sw):
            return pltpu.async_copy(
                wtab.at[widx_v.at[loff + j]],
                wrows.at[pl.ds(sw, 128)], gsem)

        fire_gather(0, 0)

        def chunk(j, carry):
            s = lax.bitwise_and(j, 1)
            sw = s * 128
            so = s * OUT_DIM

            @pl.when(j < NCH - 1)
            def _():
                fire_gather(j + 1, 128 - sw)

            pltpu.make_async_copy(
                wtab.at[widx_v.at[loff + j]],
                wrows.at[pl.ds(sw, 128)], gsem).wait()

            for db in range(8):
                bv = jnp.full((16,), sw + db * 16, jnp.int32) + lanes
                obv = jnp.full((16,), db * 16, jnp.int32) + lanes
                for dd in range(8):
                    for k in range(16):
                        c = lax.bitwise_and(lanes + k, 15)
                        colv = jnp.full((16,), dd * 16, jnp.int32) + c
                        dv = jnp.full((16,), so + dd * 16, jnp.int32) + c
                        vals = plsc.load_gather(wrows, [bv, colv])
                        plsc.store_scatter(obuf, [dv, obv], vals)

            for tab_v, idx_v, po in ((p1tab_v, p1idx_v, WORD_DIM),
                                     (p2tab_v, p2idx_v, WORD_DIM + POS_SIZE)):
                for g in range(8):
                    rr = idx_v[loff + j, pl.ds(g * 16, 16)]
                    prow = lax.shift_right_logical(rr, 3)
                    pcol = lax.shift_left(lax.bitwise_and(rr, 7), 4)
                    obv = jnp.full((16,), g * 16, jnp.int32) + lanes
                    for k in range(16):
                        c = lax.bitwise_and(lanes + k, 15)
                        vals = plsc.load_gather(tab_v, [prow, pcol + c])
                        dv = jnp.full((16,), so + po, jnp.int32) + c
                        plsc.store_scatter(obuf, [dv, obv], vals)

            @pl.when(j > 0)
            def _():
                pltpu.make_async_copy(
                    out.at[pl.ds(0, OUT_DIM), pl.ds(0, 128)],
                    obuf.at[pl.ds(OUT_DIM - so, OUT_DIM)], wsem).wait()

            gl = lread + loff + j
            pltpu.async_copy(
                obuf.at[pl.ds(so, OUT_DIM)],
                out.at[pl.ds(gl * OUT_DIM, OUT_DIM), pl.ds(col0, 128)],
                wsem)
            return carry

        lax.fori_loop(0, NCH, chunk, 0)
        pltpu.make_async_copy(out.at[pl.ds(0, OUT_DIM), pl.ds(0, 128)],
                              obuf.at[pl.ds(OUT_DIM, OUT_DIM)], wsem).wait()

    return lookup


_LOOKUP = _make_kernel()


def kernel(word, pos1, pos2, word_table, pos1_table, pos2_table):
    wf = word.astype(jnp.int32).T
    p1f = pos1.astype(jnp.int32).T
    p2f = pos2.astype(jnp.int32).T
    p1t = pos1_table.reshape(64, 128)
    p2t = pos2_table.reshape(64, 128)
    out = _LOOKUP(wf, p1f, p2f, word_table, p1t, p2t)
    return out.reshape(L, OUT_DIM, B).transpose(2, 0, 1)


# speedup vs baseline: 1.1530x; 1.1530x over previous
import functools

import jax
import jax.numpy as jnp
from jax import lax
from jax.experimental import pallas as pl
from jax.experimental.pallas import tpu as pltpu
from jax.experimental.pallas import tpu_sc as plsc

B = 1024
L = 200
WORD_DIM = 128
POS_SIZE = 16
OUT_DIM = WORD_DIM + 2 * POS_SIZE

NC, NS = 2, 16
NW = NC * NS
NBB = B // 128
NLR = NW // NBB
NCH = L // NLR
LBLK = 56


def _make_kernel():
    mesh = plsc.VectorSubcoreMesh(core_axis_name="c", subcore_axis_name="s")

    @functools.partial(
        pl.kernel,
        mesh=mesh,
        out_type=jax.ShapeDtypeStruct((L * OUT_DIM, B), jnp.float32),
        compiler_params=pltpu.CompilerParams(needs_layout_passes=False),
        scratch_types=[
            pltpu.VMEM((LBLK, 128), jnp.int32),
            pltpu.VMEM((LBLK, 128), jnp.int32),
            pltpu.VMEM((LBLK, 128), jnp.int32),
            pltpu.VMEM((64, 128), jnp.float32),
            pltpu.VMEM((64, 128), jnp.float32),
            pltpu.VMEM((2 * 128, 128), jnp.float32),
            pltpu.VMEM((2 * OUT_DIM, 128), jnp.float32),
            pltpu.SemaphoreType.DMA,
            pltpu.SemaphoreType.DMA,
        ],
    )
    def lookup(word_i, pos1_i, pos2_i, wtab, p1tab, p2tab, out,
               widx_v, p1idx_v, p2idx_v, p1tab_v, p2tab_v, wrows, obuf,
               gsem, wsem):
        wid = lax.axis_index("s") * NC + lax.axis_index("c")
        bb = lax.div(wid, NLR)
        lr = lax.rem(wid, NLR)
        col0 = bb * 128
        lread = lr * 48
        loff = lr * 2
        lanes = lax.iota(jnp.int32, 16)

        pltpu.sync_copy(word_i.at[pl.ds(lread, LBLK), pl.ds(col0, 128)],
                        widx_v)
        pltpu.sync_copy(pos1_i.at[pl.ds(lread, LBLK), pl.ds(col0, 128)],
                        p1idx_v)
        pltpu.sync_copy(pos2_i.at[pl.ds(lread, LBLK), pl.ds(col0, 128)],
                        p2idx_v)
        pltpu.sync_copy(p1tab, p1tab_v)
        pltpu.sync_copy(p2tab, p2tab_v)

        def fire_gather(j, sw):
            return pltpu.async_copy(
                wtab.at[widx_v.at[loff + j]],
                wrows.at[pl.ds(sw, 128)], gsem)

        fire_gather(0, 0)

        def chunk(j, carry):
            s = lax.bitwise_and(j, 1)
            sw = s * 128
            so = s * OUT_DIM

            @pl.when(j < NCH - 1)
            def _():
                fire_gather(j + 1, 128 - sw)

            pltpu.make_async_copy(
                wtab.at[widx_v.at[loff + j]],
                wrows.at[pl.ds(sw, 128)], gsem).wait()

            bvs = [jnp.full((16,), sw + db * 16, jnp.int32) + lanes
                   for db in range(8)]
            obvs = [jnp.full((16,), db * 16, jnp.int32) + lanes
                    for db in range(8)]
            cs = [lax.bitwise_and(lanes + k, 15) for k in range(16)]
            for dd in range(8):
                for k in range(16):
                    colv = jnp.full((16,), dd * 16, jnp.int32) + cs[k]
                    dv = jnp.full((16,), so + dd * 16, jnp.int32) + cs[k]
                    for db in range(8):
                        vals = plsc.load_gather(wrows, [bvs[db], colv])
                        plsc.store_scatter(obuf, [dv, obvs[db]], vals)

            for tab_v, idx_v, po in ((p1tab_v, p1idx_v, WORD_DIM),
                                     (p2tab_v, p2idx_v, WORD_DIM + POS_SIZE)):
                dvs = [jnp.full((16,), so + po, jnp.int32) + cs[k]
                       for k in range(16)]
                for g in range(8):
                    rr = idx_v[loff + j, pl.ds(g * 16, 16)]
                    prow = lax.shift_right_logical(rr, 3)
                    pcol = lax.shift_left(lax.bitwise_and(rr, 7), 4)
                    for k in range(16):
                        vals = plsc.load_gather(tab_v, [prow, pcol + cs[k]])
                        plsc.store_scatter(obuf, [dvs[k], obvs[g]], vals)

            @pl.when(j > 0)
            def _():
                pltpu.make_async_copy(
                    out.at[pl.ds(0, OUT_DIM), pl.ds(0, 128)],
                    obuf.at[pl.ds(OUT_DIM - so, OUT_DIM)], wsem).wait()

            gl = lread + loff + j
            pltpu.async_copy(
                obuf.at[pl.ds(so, OUT_DIM)],
                out.at[pl.ds(gl * OUT_DIM, OUT_DIM), pl.ds(col0, 128)],
                wsem)
            return carry

        lax.fori_loop(0, NCH, chunk, 0)
        pltpu.make_async_copy(out.at[pl.ds(0, OUT_DIM), pl.ds(0, 128)],
                              obuf.at[pl.ds(OUT_DIM, OUT_DIM)], wsem).wait()

    return lookup


_LOOKUP = _make_kernel()


def kernel(word, pos1, pos2, word_table, pos1_table, pos2_table):
    wf = word.astype(jnp.int32).T
    p1f = pos1.astype(jnp.int32).T
    p2f = pos2.astype(jnp.int32).T
    p1t = pos1_table.reshape(64, 128)
    p2t = pos2_table.reshape(64, 128)
    out = _LOOKUP(wf, p1f, p2f, word_table, p1t, p2t)
    return out.reshape(L, OUT_DIM, B).transpose(2, 0, 1)


# speedup vs baseline: 2.0880x; 1.8110x over previous
import functools

import jax
import jax.numpy as jnp
from jax import lax
from jax.experimental import pallas as pl
from jax.experimental.pallas import tpu as pltpu
from jax.experimental.pallas import tpu_sc as plsc

B = 1024
L = 200
WORD_DIM = 128
POS_SIZE = 16
OUT_DIM = WORD_DIM + 2 * POS_SIZE

NC, NS = 2, 16
NW = NC * NS
NBB = B // 128
NLR = NW // NBB
NCH = L // NLR
LBLK = 56


def _make_kernel():
    mesh = plsc.VectorSubcoreMesh(core_axis_name="c", subcore_axis_name="s")

    @functools.partial(
        pl.kernel,
        mesh=mesh,
        out_type=jax.ShapeDtypeStruct((L * OUT_DIM, B), jnp.float32),
        compiler_params=pltpu.CompilerParams(needs_layout_passes=False),
        scratch_types=[
            pltpu.VMEM((LBLK, 128), jnp.int32),
            pltpu.VMEM((LBLK, 128), jnp.int32),
            pltpu.VMEM((LBLK, 128), jnp.int32),
            pltpu.VMEM((64, 128), jnp.float32),
            pltpu.VMEM((64, 128), jnp.float32),
            pltpu.VMEM((2 * 128, 128), jnp.float32),
            pltpu.VMEM((2 * OUT_DIM, 128), jnp.float32),
            pltpu.SemaphoreType.DMA,
            pltpu.SemaphoreType.DMA,
        ],
    )
    def lookup(word_i, pos1_i, pos2_i, wtab, p1tab, p2tab, out,
               widx_v, p1idx_v, p2idx_v, p1tab_v, p2tab_v, wrows, obuf,
               gsem, wsem):
        wid = lax.axis_index("s") * NC + lax.axis_index("c")
        bb = lax.div(wid, NLR)
        lr = lax.rem(wid, NLR)
        col0 = bb * 128
        lread = lr * 48
        loff = lr * 2
        lanes = lax.iota(jnp.int32, 16)

        pltpu.sync_copy(word_i.at[pl.ds(lread, LBLK), pl.ds(col0, 128)],
                        widx_v)
        pltpu.sync_copy(pos1_i.at[pl.ds(lread, LBLK), pl.ds(col0, 128)],
                        p1idx_v)
        pltpu.sync_copy(pos2_i.at[pl.ds(lread, LBLK), pl.ds(col0, 128)],
                        p2idx_v)
        pltpu.sync_copy(p1tab, p1tab_v)
        pltpu.sync_copy(p2tab, p2tab_v)

        def fire_gather(j, sw):
            return pltpu.async_copy(
                wtab.at[widx_v.at[loff + j]],
                wrows.at[pl.ds(sw, 128)], gsem)

        fire_gather(0, 0)

        def chunk(j, carry):
            s = lax.bitwise_and(j, 1)
            sw = s * 128
            so = s * OUT_DIM

            @pl.when(j < NCH - 1)
            def _():
                fire_gather(j + 1, 128 - sw)

            pltpu.make_async_copy(
                wtab.at[widx_v.at[loff + j]],
                wrows.at[pl.ds(sw, 128)], gsem).wait()

            for dd in range(8):
                for k in range(16):
                    c = lax.bitwise_and(lanes + k, 15)
                    colv = jnp.full((16,), dd * 16, jnp.int32) + c
                    dv = jnp.full((16,), so + dd * 16, jnp.int32) + c
                    vals8 = []
                    for db in range(8):
                        bv = jnp.full((16,), sw + db * 16, jnp.int32) + lanes
                        vals8.append(plsc.load_gather(wrows, [bv, colv]))
                    for db in range(8):
                        obv = jnp.full((16,), db * 16, jnp.int32) + lanes
                        plsc.store_scatter(obuf, [dv, obv], vals8[db])

            for tab_v, idx_v, po in ((p1tab_v, p1idx_v, WORD_DIM),
                                     (p2tab_v, p2idx_v, WORD_DIM + POS_SIZE)):
                for g in range(8):
                    rr = idx_v[loff + j, pl.ds(g * 16, 16)]
                    prow = lax.shift_right_logical(rr, 3)
                    pcol = lax.shift_left(lax.bitwise_and(rr, 7), 4)
                    obv = jnp.full((16,), g * 16, jnp.int32) + lanes
                    vals16 = []
                    for k in range(16):
                        c = lax.bitwise_and(lanes + k, 15)
                        vals16.append(
                            plsc.load_gather(tab_v, [prow, pcol + c]))
                    for k in range(16):
                        c = lax.bitwise_and(lanes + k, 15)
                        dv = jnp.full((16,), so + po, jnp.int32) + c
                        plsc.store_scatter(obuf, [dv, obv], vals16[k])

            @pl.when(j > 0)
            def _():
                pltpu.make_async_copy(
                    out.at[pl.ds(0, OUT_DIM), pl.ds(0, 128)],
                    obuf.at[pl.ds(OUT_DIM - so, OUT_DIM)], wsem).wait()

            gl = lread + loff + j
            pltpu.async_copy(
                obuf.at[pl.ds(so, OUT_DIM)],
                out.at[pl.ds(gl * OUT_DIM, OUT_DIM), pl.ds(col0, 128)],
                wsem)
            return carry

        lax.fori_loop(0, NCH, chunk, 0)
        pltpu.make_async_copy(out.at[pl.ds(0, OUT_DIM), pl.ds(0, 128)],
                              obuf.at[pl.ds(OUT_DIM, OUT_DIM)], wsem).wait()

    return lookup


_LOOKUP = _make_kernel()


def kernel(word, pos1, pos2, word_table, pos1_table, pos2_table):
    wf = word.astype(jnp.int32).T
    p1f = pos1.astype(jnp.int32).T
    p2f = pos2.astype(jnp.int32).T
    p1t = pos1_table.reshape(64, 128)
    p2t = pos2_table.reshape(64, 128)
    out = _LOOKUP(wf, p1f, p2f, word_table, p1t, p2t)
    return out.reshape(L, OUT_DIM, B).transpose(2, 0, 1)


# speedup vs baseline: 2.1728x; 1.0406x over previous
import functools

import jax
import jax.numpy as jnp
from jax import lax
from jax.experimental import pallas as pl
from jax.experimental.pallas import tpu as pltpu
from jax.experimental.pallas import tpu_sc as plsc

B = 1024
L = 200
WORD_DIM = 128
POS_SIZE = 16
OUT_DIM = WORD_DIM + 2 * POS_SIZE

NC, NS = 2, 16
NW = NC * NS
NBB = B // 128
NLR = NW // NBB
NCH = L // NLR
LBLK = 56


def _make_kernel():
    mesh = plsc.VectorSubcoreMesh(core_axis_name="c", subcore_axis_name="s")

    @functools.partial(
        pl.kernel,
        mesh=mesh,
        out_type=jax.ShapeDtypeStruct((L * OUT_DIM, B), jnp.float32),
        compiler_params=pltpu.CompilerParams(needs_layout_passes=False),
        scratch_types=[
            pltpu.VMEM((LBLK, 128), jnp.int32),
            pltpu.VMEM((LBLK, 128), jnp.int32),
            pltpu.VMEM((LBLK, 128), jnp.int32),
            pltpu.VMEM((64, 128), jnp.float32),
            pltpu.VMEM((64, 128), jnp.float32),
            pltpu.VMEM((2 * 128, 128), jnp.float32),
            pltpu.VMEM((2 * OUT_DIM, 128), jnp.float32),
            pltpu.SemaphoreType.DMA,
            pltpu.SemaphoreType.DMA,
        ],
    )
    def lookup(word_i, pos1_i, pos2_i, wtab, p1tab, p2tab, out,
               widx_v, p1idx_v, p2idx_v, p1tab_v, p2tab_v, wrows, obuf,
               gsem, wsem):
        wid = lax.axis_index("s") * NC + lax.axis_index("c")
        bb = lax.div(wid, NLR)
        lr = lax.rem(wid, NLR)
        col0 = bb * 128
        lread = lr * 48
        loff = lr * 2
        lanes = lax.iota(jnp.int32, 16)

        pltpu.sync_copy(word_i.at[pl.ds(lread, LBLK), pl.ds(col0, 128)],
                        widx_v)
        pltpu.sync_copy(pos1_i.at[pl.ds(lread, LBLK), pl.ds(col0, 128)],
                        p1idx_v)
        pltpu.sync_copy(pos2_i.at[pl.ds(lread, LBLK), pl.ds(col0, 128)],
                        p2idx_v)
        pltpu.sync_copy(p1tab, p1tab_v)
        pltpu.sync_copy(p2tab, p2tab_v)

        def fire_gather(j, sw):
            return pltpu.async_copy(
                wtab.at[widx_v.at[loff + j]],
                wrows.at[pl.ds(sw, 128)], gsem)

        fire_gather(0, 0)

        def chunk(j, carry):
            s = lax.bitwise_and(j, 1)
            sw = s * 128
            so = s * OUT_DIM

            @pl.when(j < NCH - 1)
            def _():
                fire_gather(j + 1, 128 - sw)

            for tab_v, idx_v, po in ((p1tab_v, p1idx_v, WORD_DIM),
                                     (p2tab_v, p2idx_v, WORD_DIM + POS_SIZE)):
                for g in range(8):
                    rr = idx_v[loff + j, pl.ds(g * 16, 16)]
                    prow = lax.shift_right_logical(rr, 3)
                    pcol = lax.shift_left(lax.bitwise_and(rr, 7), 4)
                    obv = jnp.full((16,), g * 16, jnp.int32) + lanes
                    vals16 = []
                    for k in range(16):
                        c = lax.bitwise_and(lanes + k, 15)
                        vals16.append(
                            plsc.load_gather(tab_v, [prow, pcol + c]))
                    for k in range(16):
                        c = lax.bitwise_and(lanes + k, 15)
                        dv = jnp.full((16,), so + po, jnp.int32) + c
                        plsc.store_scatter(obuf, [dv, obv], vals16[k])

            pltpu.make_async_copy(
                wtab.at[widx_v.at[loff + j]],
                wrows.at[pl.ds(sw, 128)], gsem).wait()

            for dd in range(8):
                for k in range(16):
                    c = lax.bitwise_and(lanes + k, 15)
                    colv = jnp.full((16,), dd * 16, jnp.int32) + c
                    dv = jnp.full((16,), so + dd * 16, jnp.int32) + c
                    vals8 = []
                    for db in range(8):
                        bv = jnp.full((16,), sw + db * 16, jnp.int32) + lanes
                        vals8.append(plsc.load_gather(wrows, [bv, colv]))
                    for db in range(8):
                        obv = jnp.full((16,), db * 16, jnp.int32) + lanes
                        plsc.store_scatter(obuf, [dv, obv], vals8[db])

            @pl.when(j > 0)
            def _():
                pltpu.make_async_copy(
                    out.at[pl.ds(0, OUT_DIM), pl.ds(0, 128)],
                    obuf.at[pl.ds(OUT_DIM - so, OUT_DIM)], wsem).wait()

            gl = lread + loff + j
            pltpu.async_copy(
                obuf.at[pl.ds(so, OUT_DIM)],
                out.at[pl.ds(gl * OUT_DIM, OUT_DIM), pl.ds(col0, 128)],
                wsem)
            return carry

        lax.fori_loop(0, NCH, chunk, 0)
        pltpu.make_async_copy(out.at[pl.ds(0, OUT_DIM), pl.ds(0, 128)],
                              obuf.at[pl.ds(OUT_DIM, OUT_DIM)], wsem).wait()

    return lookup


_LOOKUP = _make_kernel()


def kernel(word, pos1, pos2, word_table, pos1_table, pos2_table):
    wf = word.astype(jnp.int32).T
    p1f = pos1.astype(jnp.int32).T
    p2f = pos2.astype(jnp.int32).T
    p1t = pos1_table.reshape(64, 128)
    p2t = pos2_table.reshape(64, 128)
    out = _LOOKUP(wf, p1f, p2f, word_table, p1t, p2t)
    return out.reshape(L, OUT_DIM, B).transpose(2, 0, 1)
